# Initial kernel scaffold; baseline (speedup 1.0000x reference)
import jax
import jax.numpy as jnp
from jax.experimental import pallas as pl


def kernel(x, Wq, Wk, Wv, Wo, Wr, Wg, Wu, Wd, w1, w2):
    raise NotImplementedError("write your pallas kernel here")


# speedup vs baseline: 3.9515x; 3.9515x over previous
import functools

import jax
import jax.numpy as jnp
from jax.experimental import pallas as pl
from jax.experimental.pallas import tpu as pltpu

B, S, DIM = 1, 2048, 768
HEADS, HDIM = 12, 64
INTER = 1024
E, TOPK = 64, 2
EPS = 1e-6

SBLK = 256
NSB = S // SBLK
T = 128
NA = S * TOPK
NB = NA // T + (E - 1)


def _rms(x, w):
    return w * (x * jax.lax.rsqrt(jnp.mean(x * x, axis=-1, keepdims=True) + EPS))


def _qkv_kernel(x_ref, w1_ref, wq_ref, wk_ref, wv_ref, q_ref, k_ref, v_ref):
    xn = _rms(x_ref[...], w1_ref[...])
    q_ref[...] = jnp.dot(xn, wq_ref[...], preferred_element_type=jnp.float32)
    k_ref[...] = jnp.dot(xn, wk_ref[...], preferred_element_type=jnp.float32)
    v_ref[...] = jnp.dot(xn, wv_ref[...], preferred_element_type=jnp.float32)


def _attn_kernel(q_ref, k_ref, v_ref, o_ref):
    q = q_ref[0] * (HDIM ** -0.5)
    s = jnp.dot(q, k_ref[0].T, preferred_element_type=jnp.float32)
    m = jnp.max(s, axis=-1, keepdims=True)
    p = jnp.exp(s - m)
    p = p / jnp.sum(p, axis=-1, keepdims=True)
    o_ref[0] = jnp.dot(p, v_ref[0], preferred_element_type=jnp.float32)


def _route_kernel(x_ref, a_ref, wo_ref, w2_ref, wr_ref,
                  x1_ref, h_ref, i1_ref, i2_ref, wa_ref, wb_ref):
    x1 = x_ref[...] + jnp.dot(a_ref[...], wo_ref[...],
                              preferred_element_type=jnp.float32)
    x1_ref[...] = x1
    h = _rms(x1, w2_ref[...])
    h_ref[...] = h
    logits = jnp.dot(h, wr_ref[...], preferred_element_type=jnp.float32)
    m = jnp.max(logits, axis=-1, keepdims=True)
    p = jnp.exp(logits - m)
    p = p / jnp.sum(p, axis=-1, keepdims=True)
    v1 = jnp.max(p, axis=-1)
    i1 = jnp.argmax(p, axis=-1).astype(jnp.int32)
    cols = jax.lax.broadcasted_iota(jnp.int32, (SBLK, E), 1)
    p2 = jnp.where(cols == i1[:, None], -1.0, p)
    v2 = jnp.max(p2, axis=-1)
    i2 = jnp.argmax(p2, axis=-1).astype(jnp.int32)
    tot = v1 + v2
    i1_ref[0, 0] = i1
    i2_ref[0, 0] = i2
    wa_ref[0, 0] = v1 / tot
    wb_ref[0, 0] = v2 / tot


def _moe_kernel(bexp_ref, brow_ref, bn_ref, st_ref,
                h_ref, x1_ref, sw_ref, wg_ref, wu_ref, wd_ref,
                out_ref, hs, acc):
    b = pl.program_id(0)

    @pl.when(b == 0)
    def _init():
        out_ref[...] = x1_ref[...]

    n = bn_ref[b]

    @pl.when(n > 0)
    def _work():
        base = brow_ref[b]

        def gbody(t, _):
            hs[t, :] = h_ref[st_ref[base + t], :]
            return 0
        jax.lax.fori_loop(0, T, gbody, 0, unroll=True)

        hv = hs[...]
        g = jnp.dot(hv, wg_ref[0], preferred_element_type=jnp.float32)
        u = jnp.dot(hv, wu_ref[0], preferred_element_type=jnp.float32)
        a = (g * jax.nn.sigmoid(g)) * u
        eo = jnp.dot(a, wd_ref[0], preferred_element_type=jnp.float32)

        w = sw_ref[pl.ds(base, T), :]
        rows = jax.lax.broadcasted_iota(jnp.int32, (T, 1), 0)
        w = jnp.where(rows < n, w, 0.0)
        acc[...] = eo * w

        def sbody(t, _):
            out_ref[st_ref[base + t], :] += acc[t, :]
            return 0
        jax.lax.fori_loop(0, n, sbody, 0)


def kernel(x, Wq, Wk, Wv, Wo, Wr, Wg, Wu, Wd, w1, w2):
    xf = x.reshape(S, DIM)
    w1r = w1.reshape(1, DIM)
    w2r = w2.reshape(1, DIM)

    q, k, v = pl.pallas_call(
        _qkv_kernel,
        grid=(NSB,),
        in_specs=[
            pl.BlockSpec((SBLK, DIM), lambda i: (i, 0)),
            pl.BlockSpec((1, DIM), lambda i: (0, 0)),
            pl.BlockSpec((DIM, DIM), lambda i: (0, 0)),
            pl.BlockSpec((DIM, DIM), lambda i: (0, 0)),
            pl.BlockSpec((DIM, DIM), lambda i: (0, 0)),
        ],
        out_specs=[pl.BlockSpec((SBLK, DIM), lambda i: (i, 0))] * 3,
        out_shape=[jax.ShapeDtypeStruct((S, DIM), jnp.float32)] * 3,
    )(xf, w1r, Wq, Wk, Wv)

    qh = q.reshape(S, HEADS, HDIM).transpose(1, 0, 2)
    kh = k.reshape(S, HEADS, HDIM).transpose(1, 0, 2)
    vh = v.reshape(S, HEADS, HDIM).transpose(1, 0, 2)

    attn = pl.pallas_call(
        _attn_kernel,
        grid=(HEADS, NSB),
        in_specs=[
            pl.BlockSpec((1, SBLK, HDIM), lambda h, i: (h, i, 0)),
            pl.BlockSpec((1, S, HDIM), lambda h, i: (h, 0, 0)),
            pl.BlockSpec((1, S, HDIM), lambda h, i: (h, 0, 0)),
        ],
        out_specs=pl.BlockSpec((1, SBLK, HDIM), lambda h, i: (h, i, 0)),
        out_shape=jax.ShapeDtypeStruct((HEADS, S, HDIM), jnp.float32),
    )(qh, kh, vh)
    attn = attn.transpose(1, 0, 2).reshape(S, DIM)

    x1, h, i1, i2, wa, wb = pl.pallas_call(
        _route_kernel,
        grid=(NSB,),
        in_specs=[
            pl.BlockSpec((SBLK, DIM), lambda i: (i, 0)),
            pl.BlockSpec((SBLK, DIM), lambda i: (i, 0)),
            pl.BlockSpec((DIM, DIM), lambda i: (0, 0)),
            pl.BlockSpec((1, DIM), lambda i: (0, 0)),
            pl.BlockSpec((DIM, E), lambda i: (0, 0)),
        ],
        out_specs=[
            pl.BlockSpec((SBLK, DIM), lambda i: (i, 0)),
            pl.BlockSpec((SBLK, DIM), lambda i: (i, 0)),
            pl.BlockSpec((1, 1, SBLK), lambda i: (i, 0, 0)),
            pl.BlockSpec((1, 1, SBLK), lambda i: (i, 0, 0)),
            pl.BlockSpec((1, 1, SBLK), lambda i: (i, 0, 0)),
            pl.BlockSpec((1, 1, SBLK), lambda i: (i, 0, 0)),
        ],
        out_shape=[
            jax.ShapeDtypeStruct((S, DIM), jnp.float32),
            jax.ShapeDtypeStruct((S, DIM), jnp.float32),
            jax.ShapeDtypeStruct((NSB, 1, SBLK), jnp.int32),
            jax.ShapeDtypeStruct((NSB, 1, SBLK), jnp.int32),
            jax.ShapeDtypeStruct((NSB, 1, SBLK), jnp.float32),
            jax.ShapeDtypeStruct((NSB, 1, SBLK), jnp.float32),
        ],
    )(xf, attn, Wo, w2r, Wr)

    eid = jnp.concatenate([i1.reshape(S), i2.reshape(S)])
    tok = jnp.concatenate([jnp.arange(S, dtype=jnp.int32)] * 2)
    wts = jnp.concatenate([wa.reshape(S), wb.reshape(S)])
    order = jnp.argsort(eid)
    st = jnp.concatenate([tok[order], jnp.zeros((T,), jnp.int32)])
    sw = jnp.concatenate([wts[order], jnp.zeros((T,), jnp.float32)]).reshape(NA + T, 1)

    counts = jnp.bincount(eid, length=E)
    offs = jnp.concatenate([jnp.zeros((1,), counts.dtype),
                            jnp.cumsum(counts)[:-1]])
    ntiles = (counts + T - 1) // T
    ctiles = jnp.cumsum(ntiles)
    bar = jnp.arange(NB)
    bexp = jnp.minimum(jnp.searchsorted(ctiles, bar, side="right"), E - 1)
    j = bar - (ctiles[bexp] - ntiles[bexp])
    brow = jnp.clip(offs[bexp] + j * T, 0, NA).astype(jnp.int32)
    bn = jnp.clip(counts[bexp] - j * T, 0, T).astype(jnp.int32)
    bexp = bexp.astype(jnp.int32)

    out = pl.pallas_call(
        _moe_kernel,
        grid_spec=pltpu.PrefetchScalarGridSpec(
            num_scalar_prefetch=4,
            grid=(NB,),
            in_specs=[
                pl.BlockSpec((S, DIM), lambda b, *_: (0, 0)),
                pl.BlockSpec((S, DIM), lambda b, *_: (0, 0)),
                pl.BlockSpec((NA + T, 1), lambda b, *_: (0, 0)),
                pl.BlockSpec((1, DIM, INTER), lambda b, be, br, bnn, stt: (be[b], 0, 0)),
                pl.BlockSpec((1, DIM, INTER), lambda b, be, br, bnn, stt: (be[b], 0, 0)),
                pl.BlockSpec((1, INTER, DIM), lambda b, be, br, bnn, stt: (be[b], 0, 0)),
            ],
            out_specs=pl.BlockSpec((S, DIM), lambda b, *_: (0, 0)),
            scratch_shapes=[
                pltpu.VMEM((T, DIM), jnp.float32),
                pltpu.VMEM((T, DIM), jnp.float32),
            ],
        ),
        out_shape=jax.ShapeDtypeStruct((S, DIM), jnp.float32),
        compiler_params=pltpu.CompilerParams(
            dimension_semantics=("arbitrary",),
        ),
    )(bexp, brow, bn, st, h, x1, sw, Wg, Wu, Wd)

    return out.reshape(B, S, DIM)


# speedup vs baseline: 4.9115x; 1.2429x over previous
import functools

import jax
import jax.numpy as jnp
from jax.experimental import pallas as pl
from jax.experimental.pallas import tpu as pltpu

B, S, DIM = 1, 2048, 768
HEADS, HDIM = 12, 64
INTER = 1024
E, TOPK = 64, 2
EPS = 1e-6

SBLK = 256
NSB = S // SBLK
T = 128
NA = S * TOPK
NB = NA // T + (E - 1)


def _rms(x, w):
    return w * (x * jax.lax.rsqrt(jnp.mean(x * x, axis=-1, keepdims=True) + EPS))


def _bdot(a, b):
    return jnp.dot(a.astype(jnp.bfloat16), b.astype(jnp.bfloat16),
                   preferred_element_type=jnp.float32)


def _qkv_kernel(x_ref, w1_ref, wq_ref, wk_ref, wv_ref, q_ref, k_ref, v_ref):
    xn = _rms(x_ref[...], w1_ref[...])
    q_ref[...] = _bdot(xn, wq_ref[...])
    k_ref[...] = _bdot(xn, wk_ref[...])
    v_ref[...] = _bdot(xn, wv_ref[...])


def _attn_kernel(q_ref, k_ref, v_ref, o_ref):
    for hp in range(2):
        sl = slice(hp * HDIM, (hp + 1) * HDIM)
        q = (q_ref[:, sl] * (HDIM ** -0.5)).astype(jnp.bfloat16)
        k = k_ref[:, sl].astype(jnp.bfloat16)
        s = jax.lax.dot_general(q, k, (((1,), (1,)), ((), ())),
                                preferred_element_type=jnp.float32)
        m = jnp.max(s, axis=-1, keepdims=True)
        p = jnp.exp(s - m)
        p = p / jnp.sum(p, axis=-1, keepdims=True)
        o_ref[:, sl] = _bdot(p, v_ref[:, sl])


def _route_kernel(x_ref, a_ref, wo_ref, w2_ref, wr_ref,
                  x1_ref, h_ref, i1_ref, i2_ref, wa_ref, wb_ref):
    x1 = x_ref[...] + _bdot(a_ref[...], wo_ref[...])
    x1_ref[...] = x1
    h = _rms(x1, w2_ref[...])
    h_ref[...] = h
    logits = jnp.dot(h, wr_ref[...], preferred_element_type=jnp.float32)
    m = jnp.max(logits, axis=-1, keepdims=True)
    p = jnp.exp(logits - m)
    p = p / jnp.sum(p, axis=-1, keepdims=True)
    v1 = jnp.max(p, axis=-1)
    i1 = jnp.argmax(p, axis=-1).astype(jnp.int32)
    cols = jax.lax.broadcasted_iota(jnp.int32, (SBLK, E), 1)
    p2 = jnp.where(cols == i1[:, None], -1.0, p)
    v2 = jnp.max(p2, axis=-1)
    i2 = jnp.argmax(p2, axis=-1).astype(jnp.int32)
    tot = v1 + v2
    i1_ref[0, 0] = i1
    i2_ref[0, 0] = i2
    wa_ref[0, 0] = v1 / tot
    wb_ref[0, 0] = v2 / tot


def _moe_kernel(bexp_ref, brow_ref, bn_ref, st_ref,
                h_ref, x1_ref, sw_ref, wg_ref, wu_ref, wd_ref,
                out_ref, hs, acc):
    b = pl.program_id(0)

    @pl.when(b == 0)
    def _init():
        out_ref[...] = x1_ref[...]

    n = bn_ref[b]

    @pl.when(n > 0)
    def _work():
        base = brow_ref[b]

        def gbody(t, _):
            hs[t, :] = h_ref[st_ref[base + t], :]
            return 0
        jax.lax.fori_loop(0, T, gbody, 0, unroll=True)

        hv = hs[...]
        g = _bdot(hv, wg_ref[0])
        u = _bdot(hv, wu_ref[0])
        a = (g * jax.nn.sigmoid(g)) * u
        eo = _bdot(a, wd_ref[0])

        w = sw_ref[pl.ds(base, T), :]
        rows = jax.lax.broadcasted_iota(jnp.int32, (T, 1), 0)
        w = jnp.where(rows < n, w, 0.0)
        acc[...] = eo * w

        def sbody(t, _):
            out_ref[st_ref[base + t], :] += acc[t, :]
            return 0
        jax.lax.fori_loop(0, n, sbody, 0)


def kernel(x, Wq, Wk, Wv, Wo, Wr, Wg, Wu, Wd, w1, w2):
    xf = x.reshape(S, DIM)
    w1r = w1.reshape(1, DIM)
    w2r = w2.reshape(1, DIM)

    q, k, v = pl.pallas_call(
        _qkv_kernel,
        grid=(NSB,),
        in_specs=[
            pl.BlockSpec((SBLK, DIM), lambda i: (i, 0)),
            pl.BlockSpec((1, DIM), lambda i: (0, 0)),
            pl.BlockSpec((DIM, DIM), lambda i: (0, 0)),
            pl.BlockSpec((DIM, DIM), lambda i: (0, 0)),
            pl.BlockSpec((DIM, DIM), lambda i: (0, 0)),
        ],
        out_specs=[pl.BlockSpec((SBLK, DIM), lambda i: (i, 0))] * 3,
        out_shape=[jax.ShapeDtypeStruct((S, DIM), jnp.float32)] * 3,
    )(xf, w1r, Wq, Wk, Wv)

    attn = pl.pallas_call(
        _attn_kernel,
        grid=(HEADS // 2, NSB),
        in_specs=[
            pl.BlockSpec((SBLK, 2 * HDIM), lambda h, i: (i, h)),
            pl.BlockSpec((S, 2 * HDIM), lambda h, i: (0, h)),
            pl.BlockSpec((S, 2 * HDIM), lambda h, i: (0, h)),
        ],
        out_specs=pl.BlockSpec((SBLK, 2 * HDIM), lambda h, i: (i, h)),
        out_shape=jax.ShapeDtypeStruct((S, DIM), jnp.float32),
    )(q, k, v)

    x1, h, i1, i2, wa, wb = pl.pallas_call(
        _route_kernel,
        grid=(NSB,),
        in_specs=[
            pl.BlockSpec((SBLK, DIM), lambda i: (i, 0)),
            pl.BlockSpec((SBLK, DIM), lambda i: (i, 0)),
            pl.BlockSpec((DIM, DIM), lambda i: (0, 0)),
            pl.BlockSpec((1, DIM), lambda i: (0, 0)),
            pl.BlockSpec((DIM, E), lambda i: (0, 0)),
        ],
        out_specs=[
            pl.BlockSpec((SBLK, DIM), lambda i: (i, 0)),
            pl.BlockSpec((SBLK, DIM), lambda i: (i, 0)),
            pl.BlockSpec((1, 1, SBLK), lambda i: (i, 0, 0)),
            pl.BlockSpec((1, 1, SBLK), lambda i: (i, 0, 0)),
            pl.BlockSpec((1, 1, SBLK), lambda i: (i, 0, 0)),
            pl.BlockSpec((1, 1, SBLK), lambda i: (i, 0, 0)),
        ],
        out_shape=[
            jax.ShapeDtypeStruct((S, DIM), jnp.float32),
            jax.ShapeDtypeStruct((S, DIM), jnp.float32),
            jax.ShapeDtypeStruct((NSB, 1, SBLK), jnp.int32),
            jax.ShapeDtypeStruct((NSB, 1, SBLK), jnp.int32),
            jax.ShapeDtypeStruct((NSB, 1, SBLK), jnp.float32),
            jax.ShapeDtypeStruct((NSB, 1, SBLK), jnp.float32),
        ],
    )(xf, attn, Wo, w2r, Wr)

    eid = jnp.concatenate([i1.reshape(S), i2.reshape(S)])
    tok = jnp.concatenate([jnp.arange(S, dtype=jnp.int32)] * 2)
    wts = jnp.concatenate([wa.reshape(S), wb.reshape(S)])
    order = jnp.argsort(eid)
    st = jnp.concatenate([tok[order], jnp.zeros((T,), jnp.int32)])
    sw = jnp.concatenate([wts[order], jnp.zeros((T,), jnp.float32)]).reshape(NA + T, 1)

    counts = jnp.bincount(eid, length=E)
    offs = jnp.concatenate([jnp.zeros((1,), counts.dtype),
                            jnp.cumsum(counts)[:-1]])
    ntiles = (counts + T - 1) // T
    ctiles = jnp.cumsum(ntiles)
    bar = jnp.arange(NB)
    bexp = jnp.minimum(jnp.searchsorted(ctiles, bar, side="right"), E - 1)
    j = bar - (ctiles[bexp] - ntiles[bexp])
    brow = jnp.clip(offs[bexp] + j * T, 0, NA).astype(jnp.int32)
    bn = jnp.clip(counts[bexp] - j * T, 0, T).astype(jnp.int32)
    bexp = bexp.astype(jnp.int32)

    out = pl.pallas_call(
        _moe_kernel,
        grid_spec=pltpu.PrefetchScalarGridSpec(
            num_scalar_prefetch=4,
            grid=(NB,),
            in_specs=[
                pl.BlockSpec((S, DIM), lambda b, *_: (0, 0)),
                pl.BlockSpec((S, DIM), lambda b, *_: (0, 0)),
                pl.BlockSpec((NA + T, 1), lambda b, *_: (0, 0)),
                pl.BlockSpec((1, DIM, INTER), lambda b, be, br, bnn, stt: (be[b], 0, 0)),
                pl.BlockSpec((1, DIM, INTER), lambda b, be, br, bnn, stt: (be[b], 0, 0)),
                pl.BlockSpec((1, INTER, DIM), lambda b, be, br, bnn, stt: (be[b], 0, 0)),
            ],
            out_specs=pl.BlockSpec((S, DIM), lambda b, *_: (0, 0)),
            scratch_shapes=[
                pltpu.VMEM((T, DIM), jnp.float32),
                pltpu.VMEM((T, DIM), jnp.float32),
            ],
        ),
        out_shape=jax.ShapeDtypeStruct((S, DIM), jnp.float32),
        compiler_params=pltpu.CompilerParams(
            dimension_semantics=("arbitrary",),
        ),
    )(bexp, brow, bn, st, h, x1, sw, Wg, Wu, Wd)

    return out.reshape(B, S, DIM)
